# Initial kernel scaffold; baseline (speedup 1.0000x reference)
import jax
import jax.numpy as jnp
from jax.experimental import pallas as pl


def kernel(img_pos, img_neg, img_grad, scores):
    raise NotImplementedError("write your pallas kernel here")


# speedup vs baseline: 1.0666x; 1.0666x over previous
import jax
import jax.numpy as jnp
from jax.experimental import pallas as pl

B, R, D = 128, 28, 2048
K = 7
REM = R - K


def _ranks(g):
    nsq = jnp.sum(g * g, axis=1, keepdims=True)
    lt = nsq.T < nsq
    eq = nsq.T == nsq
    ir = jax.lax.broadcasted_iota(jnp.int32, (R, R), 0)
    is_ = jax.lax.broadcasted_iota(jnp.int32, (R, R), 1)
    tie = eq & (is_ < ir)
    return jnp.sum((lt | tie).astype(jnp.int32), axis=1, keepdims=True)


def _main_kernel(gpos_ref, gneg_ref, pos_ref, neg_ref,
                 syn_ref, posm_ref, negm_ref):
    gp = gpos_ref[0]
    gn = gneg_ref[0]
    pos = pos_ref[0]
    neg = neg_ref[0]

    rp = _ranks(gp)
    rn = _ranks(gn)
    top_p = rp >= REM
    top_n = rn >= REM

    sel = ((rp == rn.T) & top_p).astype(jnp.float32)
    gathered = jnp.dot(sel, neg, preferred_element_type=jnp.float32,
                       precision=jax.lax.Precision.HIGHEST)
    syn_ref[0] = jnp.where(top_p, gathered, pos)

    mean_p = jnp.sum(jnp.where(top_p, 0.0, pos), axis=0, keepdims=True) / REM
    posm_ref[0] = jnp.where(top_p, mean_p, pos)
    mean_n = jnp.sum(jnp.where(top_n, 0.0, neg), axis=0, keepdims=True) / REM
    negm_ref[0] = jnp.where(top_n, mean_n, neg)


def _argmax_kernel(s_ref, cap_ref, imgn_ref):
    s = s_ref[...]
    ir = jax.lax.broadcasted_iota(jnp.int32, (B, B), 0)
    ic = jax.lax.broadcasted_iota(jnp.int32, (B, B), 1)
    s2 = jnp.where(ir == ic, s - 10.0, s)
    m1 = jnp.max(s2, axis=1, keepdims=True)
    cap_ref[...] = jnp.min(jnp.where(s2 == m1, ic, B), axis=1, keepdims=True)
    m0 = jnp.max(s2, axis=0, keepdims=True)
    imgn_ref[...] = jnp.min(jnp.where(s2 == m0, ir, B), axis=0, keepdims=True)


def kernel(img_pos, img_neg, img_grad, scores):
    blk = pl.BlockSpec((1, R, D), lambda i: (i, 0, 0))
    gblk = pl.BlockSpec((1, R, D), lambda i: (i, 0, 0))
    gblk2 = pl.BlockSpec((1, R, D), lambda i: (i + B, 0, 0))
    syn, posm, negm = pl.pallas_call(
        _main_kernel,
        grid=(B,),
        in_specs=[gblk, gblk2, blk, blk],
        out_specs=[blk, blk, blk],
        out_shape=[jax.ShapeDtypeStruct((B, R, D), jnp.float32)] * 3,
    )(img_grad, img_grad, img_pos, img_neg)

    cap, imgn = pl.pallas_call(
        _argmax_kernel,
        out_shape=[jax.ShapeDtypeStruct((B, 1), jnp.int32),
                   jax.ShapeDtypeStruct((1, B), jnp.int32)],
    )(scores)
    return syn, posm, negm, cap.reshape(B), imgn.reshape(B)


# speedup vs baseline: 1.3294x; 1.2464x over previous
import jax
import jax.numpy as jnp
from jax.experimental import pallas as pl

B, R, D = 128, 28, 2048
K = 7
REM = R - K


def _ranks(g):
    nsq = jnp.sum(g * g, axis=1, keepdims=True)
    lt = nsq.T < nsq
    eq = nsq.T == nsq
    ir = jax.lax.broadcasted_iota(jnp.int32, (R, R), 0)
    is_ = jax.lax.broadcasted_iota(jnp.int32, (R, R), 1)
    tie = eq & (is_ < ir)
    return jnp.sum((lt | tie).astype(jnp.int32), axis=1, keepdims=True)


BB = 8


def _main_kernel(gpos_ref, gneg_ref, pos_ref, neg_ref,
                 syn_ref, posm_ref, negm_ref):
    for b in range(BB):
        gp = gpos_ref[b]
        gn = gneg_ref[b]
        pos = pos_ref[b]
        neg = neg_ref[b]

        rp = _ranks(gp)
        rn = _ranks(gn)
        top_p = rp >= REM
        top_n = rn >= REM

        sel = ((rp == rn.T) & top_p).astype(jnp.float32)
        gathered = jnp.dot(sel, neg, preferred_element_type=jnp.float32,
                           precision=jax.lax.Precision.HIGHEST)
        syn_ref[b] = jnp.where(top_p, gathered, pos)

        mean_p = jnp.sum(jnp.where(top_p, 0.0, pos), axis=0,
                         keepdims=True) / REM
        posm_ref[b] = jnp.where(top_p, mean_p, pos)
        mean_n = jnp.sum(jnp.where(top_n, 0.0, neg), axis=0,
                         keepdims=True) / REM
        negm_ref[b] = jnp.where(top_n, mean_n, neg)


def _argmax_kernel(s_ref, cap_ref, imgn_ref):
    s = s_ref[...]
    ir = jax.lax.broadcasted_iota(jnp.int32, (B, B), 0)
    ic = jax.lax.broadcasted_iota(jnp.int32, (B, B), 1)
    s2 = jnp.where(ir == ic, s - 10.0, s)
    m1 = jnp.max(s2, axis=1, keepdims=True)
    cap_ref[...] = jnp.min(jnp.where(s2 == m1, ic, B), axis=1, keepdims=True)
    m0 = jnp.max(s2, axis=0, keepdims=True)
    imgn_ref[...] = jnp.min(jnp.where(s2 == m0, ir, B), axis=0, keepdims=True)


def kernel(img_pos, img_neg, img_grad, scores):
    blk = pl.BlockSpec((BB, R, D), lambda i: (i, 0, 0))
    gblk = pl.BlockSpec((BB, R, D), lambda i: (i, 0, 0))
    gblk2 = pl.BlockSpec((BB, R, D), lambda i: (i + B // BB, 0, 0))
    syn, posm, negm = pl.pallas_call(
        _main_kernel,
        grid=(B // BB,),
        in_specs=[gblk, gblk2, blk, blk],
        out_specs=[blk, blk, blk],
        out_shape=[jax.ShapeDtypeStruct((B, R, D), jnp.float32)] * 3,
    )(img_grad, img_grad, img_pos, img_neg)

    cap, imgn = pl.pallas_call(
        _argmax_kernel,
        out_shape=[jax.ShapeDtypeStruct((B, 1), jnp.int32),
                   jax.ShapeDtypeStruct((1, B), jnp.int32)],
    )(scores)
    return syn, posm, negm, cap.reshape(B), imgn.reshape(B)


# speedup vs baseline: 1.3327x; 1.0025x over previous
import jax
import jax.numpy as jnp
from jax.experimental import pallas as pl

B, R, D = 128, 28, 2048
K = 7
REM = R - K


def _ranks(g):
    nsq = jnp.sum(g * g, axis=1, keepdims=True)
    lt = nsq.T < nsq
    eq = nsq.T == nsq
    ir = jax.lax.broadcasted_iota(jnp.int32, (R, R), 0)
    is_ = jax.lax.broadcasted_iota(jnp.int32, (R, R), 1)
    tie = eq & (is_ < ir)
    return jnp.sum((lt | tie).astype(jnp.int32), axis=1, keepdims=True)


BB = 16


def _main_kernel(gpos_ref, gneg_ref, pos_ref, neg_ref,
                 syn_ref, posm_ref, negm_ref):
    for b in range(BB):
        gp = gpos_ref[b]
        gn = gneg_ref[b]
        pos = pos_ref[b]
        neg = neg_ref[b]

        rp = _ranks(gp)
        rn = _ranks(gn)
        top_p = rp >= REM
        top_n = rn >= REM

        sel = ((rp == rn.T) & top_p).astype(jnp.float32)
        gathered = jnp.dot(sel, neg, preferred_element_type=jnp.float32,
                           precision=jax.lax.Precision.HIGHEST)
        syn_ref[b] = jnp.where(top_p, gathered, pos)

        mean_p = jnp.sum(jnp.where(top_p, 0.0, pos), axis=0,
                         keepdims=True) / REM
        posm_ref[b] = jnp.where(top_p, mean_p, pos)
        mean_n = jnp.sum(jnp.where(top_n, 0.0, neg), axis=0,
                         keepdims=True) / REM
        negm_ref[b] = jnp.where(top_n, mean_n, neg)


def _argmax_kernel(s_ref, cap_ref, imgn_ref):
    s = s_ref[...]
    ir = jax.lax.broadcasted_iota(jnp.int32, (B, B), 0)
    ic = jax.lax.broadcasted_iota(jnp.int32, (B, B), 1)
    s2 = jnp.where(ir == ic, s - 10.0, s)
    m1 = jnp.max(s2, axis=1, keepdims=True)
    cap_ref[...] = jnp.min(jnp.where(s2 == m1, ic, B), axis=1, keepdims=True)
    m0 = jnp.max(s2, axis=0, keepdims=True)
    imgn_ref[...] = jnp.min(jnp.where(s2 == m0, ir, B), axis=0, keepdims=True)


def kernel(img_pos, img_neg, img_grad, scores):
    blk = pl.BlockSpec((BB, R, D), lambda i: (i, 0, 0))
    gblk = pl.BlockSpec((BB, R, D), lambda i: (i, 0, 0))
    gblk2 = pl.BlockSpec((BB, R, D), lambda i: (i + B // BB, 0, 0))
    syn, posm, negm = pl.pallas_call(
        _main_kernel,
        grid=(B // BB,),
        in_specs=[gblk, gblk2, blk, blk],
        out_specs=[blk, blk, blk],
        out_shape=[jax.ShapeDtypeStruct((B, R, D), jnp.float32)] * 3,
    )(img_grad, img_grad, img_pos, img_neg)

    cap, imgn = pl.pallas_call(
        _argmax_kernel,
        out_shape=[jax.ShapeDtypeStruct((B, 1), jnp.int32),
                   jax.ShapeDtypeStruct((1, B), jnp.int32)],
    )(scores)
    return syn, posm, negm, cap.reshape(B), imgn.reshape(B)
